# baseline (speedup 1.0000x reference)
import functools

import jax
import jax.numpy as jnp
import numpy as np
from jax import lax
from jax.experimental import pallas as pl
from jax.experimental.pallas import tpu as pltpu
from jax.experimental.pallas import tpu_sc as plsc

K = 64
B = 128
H = 32768
NC, NS, L = 2, 16, 16
NW = NC * NS
ROWS_PER_W = B // NW
NVREG = H // L
HEAD_VREGS = 16
CHUNK = 127
NCHUNK = (NVREG - HEAD_VREGS) // CHUNK
REBUILD_AT = 560
CAP = 2624

INT_MIN = np.int32(-2147483648)
MANT = np.int32(0x7FFFFFFF)

parallel_loop = plsc.parallel_loop


def _iota():
    return lax.iota(jnp.int32, L)


def _pcount(mask):
    return plsc.all_reduce_population_count(mask)[0]


def _keys_of(v):
    u = plsc.bitcast(v, jnp.int32)
    return u ^ ((u >> 31) & MANT)


def _axidx(name):
    return lax.axis_index(name)


def _body(x_hbm, o_hbm, buf0, buf1, zbuf, winbuf, cand_i, cand_k,
          sem_in, sem_out):
    wid = _axidx("c") * NS + _axidx("s")
    zeros = jnp.zeros((L,), jnp.float32)
    bufs = [buf0, buf1]

    @parallel_loop(0, H, L, unroll=8, carry=jnp.int32(0))
    def _z(off, c):
        zbuf[pl.ds(off, L)] = zeros
        return c

    def rebuild(buf, cnt):
        mv = (cnt + L - 1) // L

        @parallel_loop(0, mv * L, L, unroll=2, carry=jnp.int32(0))
        def _keys(off, c):
            idx = cand_i[pl.ds(off, L)]
            valid = (_iota() + off) < cnt
            idxs = jnp.where(valid, idx, 0)
            v = plsc.load_gather(buf, [idxs])
            k = jnp.where(valid, _keys_of(v), INT_MIN)
            cand_k[pl.ds(off, L)] = k
            return c

        def search_it(_, lohi):
            lo, hi = lohi
            mid = (lo >> 1) + (hi >> 1) + ((lo & 1) | (hi & 1))

            @parallel_loop(0, mv * L, L, unroll=4,
                           carry=jnp.zeros((L,), jnp.int32))
            def acc(off, a):
                k = cand_k[pl.ds(off, L)]
                return a + (k >= mid).astype(jnp.int32)
            pred = jnp.sum(acc) >= K
            lo = jnp.where(pred, mid, lo)
            hi = jnp.where(pred, hi, mid - jnp.int32(1))
            return lo, hi
        v_key, _ = lax.fori_loop(
            0, 32, search_it, (INT_MIN, jnp.int32(2147483647)))

        @parallel_loop(0, mv * L, L, unroll=4,
                       carry=jnp.zeros((L,), jnp.int32))
        def gacc(off, a):
            k = cand_k[pl.ds(off, L)]
            return a + (k > v_key).astype(jnp.int32)
        cnt_gt = jnp.sum(gacc)
        allow = jnp.int32(K) - cnt_gt

        def comp_j(j, carry):
            oc, tr = carry
            k = cand_k[pl.ds(j * L, L)]
            idx = cand_i[pl.ds(j * L, L)]
            gt = k > v_key
            tie = k == v_key
            tp = plsc.cumsum(tie.astype(jnp.int32)) + tr
            keep = gt | (tie & (tp <= allow))
            plsc.store_compressed(cand_i.at[pl.ds(oc, L)], idx, mask=keep)
            return oc + _pcount(keep), tr + _pcount(tie)
        lax.fori_loop(0, mv, comp_j, (jnp.int32(0), jnp.int32(0)))

        thr_bits = jnp.full((L,), v_key ^ ((v_key >> 31) & MANT), jnp.int32)
        return plsc.bitcast(thr_bits, jnp.float32)

    def select_row(buf):
        with jax.named_scope("seed_rb"):
            @parallel_loop(0, HEAD_VREGS * L, L, unroll=4, carry=jnp.int32(0))
            def _seed(off, c):
                cand_i[pl.ds(off, L)] = _iota() + off
                return c
            thr = rebuild(buf, jnp.int32(HEAD_VREGS * L))
        cnt = jnp.int32(K)

        def chunk_c(c, carry):
            cnt, thr = carry
            base = HEAD_VREGS + c * CHUNK

            @parallel_loop(base * L, (base + CHUNK) * L, L,
                           unroll=4, carry=cnt)
            def cnt(off, cnt):
                v = buf[pl.ds(off, L)]
                m = v > thr
                plsc.store_compressed(
                    cand_i.at[pl.ds(cnt, L)], _iota() + off, mask=m)
                return cnt + _pcount(m)

            def do_rb(cnt):
                return jnp.int32(K), rebuild(buf, cnt)

            cnt, thr = lax.cond(
                cnt > REBUILD_AT, do_rb, lambda c: (c, thr), cnt)
            return cnt, thr
        with jax.named_scope("filter"):
            cnt, _ = lax.fori_loop(0, NCHUNK, chunk_c, (cnt, thr))

        with jax.named_scope("final_rb"):
            rebuild(buf, cnt)

    base_row = wid * ROWS_PER_W
    pltpu.async_copy(x_hbm.at[base_row], bufs[0], sem_in)
    for t in range(ROWS_PER_W):
        buf = bufs[t % 2]
        row = base_row + t
        pltpu.make_async_copy(x_hbm.at[row], buf, sem_in).wait()
        if t + 1 < ROWS_PER_W:
            pltpu.async_copy(x_hbm.at[row + 1], bufs[(t + 1) % 2], sem_in)

        select_row(buf)

        if t >= 1:
            pltpu.make_async_copy(zbuf, o_hbm.at[row - 1], sem_out).wait()
            for j in range(K // L):
                idx = winbuf[pl.ds(j * L, L)]
                plsc.store_scatter(zbuf, [idx], zeros)
        for j in range(K // L):
            idx = cand_i[pl.ds(j * L, L)]
            vals = plsc.load_gather(buf, [idx])
            plsc.store_scatter(zbuf, [idx], vals)
            winbuf[pl.ds(j * L, L)] = idx
        pltpu.async_copy(zbuf, o_hbm.at[row], sem_out)
    pltpu.make_async_copy(
        zbuf, o_hbm.at[base_row + ROWS_PER_W - 1], sem_out).wait()


@jax.jit
def kernel(hidden_preactivation_BH):
    mesh = plsc.VectorSubcoreMesh(
        core_axis_name="c", subcore_axis_name="s",
        num_cores=NC, num_subcores=NS)
    return pl.kernel(
        _body,
        out_type=jax.ShapeDtypeStruct((B, H), jnp.float32),
        mesh=mesh,
        scratch_types=[
            pltpu.VMEM((H,), jnp.float32),
            pltpu.VMEM((H,), jnp.float32),
            pltpu.VMEM((H,), jnp.float32),
            pltpu.VMEM((K,), jnp.int32),
            pltpu.VMEM((CAP,), jnp.int32),
            pltpu.VMEM((CAP,), jnp.int32),
            pltpu.SemaphoreType.DMA,
            pltpu.SemaphoreType.DMA,
        ],
        compiler_params=pltpu.CompilerParams(needs_layout_passes=False),
    )(hidden_preactivation_BH)


# speedup vs baseline: 1.0145x; 1.0145x over previous
import functools

import jax
import jax.numpy as jnp
import numpy as np
from jax import lax
from jax.experimental import pallas as pl
from jax.experimental.pallas import tpu as pltpu
from jax.experimental.pallas import tpu_sc as plsc

K = 64
B = 128
H = 32768
NC, NS, L = 2, 16, 16
NW = NC * NS
ROWS_PER_W = B // NW
NVREG = H // L
HEAD_VREGS = 16
CHUNK = 127
NCHUNK = (NVREG - HEAD_VREGS) // CHUNK
REBUILD_AT = 560
CAP = 2624

INT_MIN = np.int32(-2147483648)
MANT = np.int32(0x7FFFFFFF)

parallel_loop = plsc.parallel_loop


def _iota():
    return lax.iota(jnp.int32, L)


def _pcount(mask):
    return plsc.all_reduce_population_count(mask)[0]


def _keys_of(v):
    u = plsc.bitcast(v, jnp.int32)
    return u ^ ((u >> 31) & MANT)


def _axidx(name):
    return lax.axis_index(name)


def _body(x_hbm, o_hbm, buf0, buf1, zbuf, winbuf, cand_i, cand_k,
          sem_in, sem_out):
    wid = _axidx("c") * NS + _axidx("s")
    zeros = jnp.zeros((L,), jnp.float32)
    bufs = [buf0, buf1]

    @parallel_loop(0, H, L, unroll=8, carry=jnp.int32(0))
    def _z(off, c):
        zbuf[pl.ds(off, L)] = zeros
        return c

    def rebuild(buf, cnt):
        mv = (cnt + L - 1) // L

        @parallel_loop(0, mv * L, L, unroll=2, carry=jnp.int32(0))
        def _keys(off, c):
            idx = cand_i[pl.ds(off, L)]
            valid = (_iota() + off) < cnt
            idxs = jnp.where(valid, idx, 0)
            v = plsc.load_gather(buf, [idxs])
            k = jnp.where(valid, _keys_of(v), INT_MIN)
            cand_k[pl.ds(off, L)] = k
            return c

        def search_it(_, lohi):
            lo, hi = lohi
            mid = (lo >> 1) + (hi >> 1) + ((lo & 1) | (hi & 1))

            @parallel_loop(0, mv * L, L, unroll=4,
                           carry=jnp.zeros((L,), jnp.int32))
            def acc(off, a):
                k = cand_k[pl.ds(off, L)]
                return a + (k >= mid).astype(jnp.int32)
            pred = jnp.sum(acc) >= K
            lo = jnp.where(pred, mid, lo)
            hi = jnp.where(pred, hi, mid - jnp.int32(1))
            return lo, hi
        v_key, _ = lax.fori_loop(
            0, 32, search_it, (INT_MIN, jnp.int32(2147483647)))

        @parallel_loop(0, mv * L, L, unroll=4,
                       carry=jnp.zeros((L,), jnp.int32))
        def gacc(off, a):
            k = cand_k[pl.ds(off, L)]
            return a + (k > v_key).astype(jnp.int32)
        cnt_gt = jnp.sum(gacc)
        allow = jnp.int32(K) - cnt_gt

        def comp_j(j, carry):
            oc, tr = carry
            k = cand_k[pl.ds(j * L, L)]
            idx = cand_i[pl.ds(j * L, L)]
            gt = k > v_key
            tie = k == v_key
            tp = plsc.cumsum(tie.astype(jnp.int32)) + tr
            keep = gt | (tie & (tp <= allow))
            plsc.store_compressed(cand_i.at[pl.ds(oc, L)], idx, mask=keep)
            return oc + _pcount(keep), tr + _pcount(tie)
        lax.fori_loop(0, mv, comp_j, (jnp.int32(0), jnp.int32(0)))

        thr_bits = jnp.full((L,), v_key ^ ((v_key >> 31) & MANT), jnp.int32)
        return plsc.bitcast(thr_bits, jnp.float32)

    def select_row(buf):
        with jax.named_scope("seed_rb"):
            @parallel_loop(0, HEAD_VREGS * L, L, unroll=4, carry=jnp.int32(0))
            def _seed(off, c):
                cand_i[pl.ds(off, L)] = _iota() + off
                return c
            thr = rebuild(buf, jnp.int32(HEAD_VREGS * L))
        cnt = jnp.int32(K)

        def chunk_c(c, carry):
            cnt, thr = carry
            base = HEAD_VREGS + c * CHUNK

            @parallel_loop(base * L, (base + CHUNK) * L, L,
                           unroll=4, carry=jnp.full((L,), cnt - 1, jnp.int32))
            def cm1(off, cm1):
                v = buf[pl.ds(off, L)]
                m = v > thr
                sc = plsc.cumsum(m.astype(jnp.int32))
                plsc.store_scatter(cand_i, [cm1 + sc], _iota() + off, mask=m)
                return cm1 + plsc.all_reduce_population_count(m)
            cnt = cm1[0] + 1

            def do_rb(cnt):
                return jnp.int32(K), rebuild(buf, cnt)

            cnt, thr = lax.cond(
                cnt > REBUILD_AT, do_rb, lambda c: (c, thr), cnt)
            return cnt, thr
        with jax.named_scope("filter"):
            cnt, _ = lax.fori_loop(0, NCHUNK, chunk_c, (cnt, thr))

        with jax.named_scope("final_rb"):
            rebuild(buf, cnt)

    base_row = wid * ROWS_PER_W
    pltpu.async_copy(x_hbm.at[base_row], bufs[0], sem_in)
    for t in range(ROWS_PER_W):
        buf = bufs[t % 2]
        row = base_row + t
        pltpu.make_async_copy(x_hbm.at[row], buf, sem_in).wait()
        if t + 1 < ROWS_PER_W:
            pltpu.async_copy(x_hbm.at[row + 1], bufs[(t + 1) % 2], sem_in)

        select_row(buf)

        if t >= 1:
            pltpu.make_async_copy(zbuf, o_hbm.at[row - 1], sem_out).wait()
            for j in range(K // L):
                idx = winbuf[pl.ds(j * L, L)]
                plsc.store_scatter(zbuf, [idx], zeros)
        for j in range(K // L):
            idx = cand_i[pl.ds(j * L, L)]
            vals = plsc.load_gather(buf, [idx])
            plsc.store_scatter(zbuf, [idx], vals)
            winbuf[pl.ds(j * L, L)] = idx
        pltpu.async_copy(zbuf, o_hbm.at[row], sem_out)
    pltpu.make_async_copy(
        zbuf, o_hbm.at[base_row + ROWS_PER_W - 1], sem_out).wait()


@jax.jit
def kernel(hidden_preactivation_BH):
    mesh = plsc.VectorSubcoreMesh(
        core_axis_name="c", subcore_axis_name="s",
        num_cores=NC, num_subcores=NS)
    return pl.kernel(
        _body,
        out_type=jax.ShapeDtypeStruct((B, H), jnp.float32),
        mesh=mesh,
        scratch_types=[
            pltpu.VMEM((H,), jnp.float32),
            pltpu.VMEM((H,), jnp.float32),
            pltpu.VMEM((H,), jnp.float32),
            pltpu.VMEM((K,), jnp.int32),
            pltpu.VMEM((CAP,), jnp.int32),
            pltpu.VMEM((CAP,), jnp.int32),
            pltpu.SemaphoreType.DMA,
            pltpu.SemaphoreType.DMA,
        ],
        compiler_params=pltpu.CompilerParams(needs_layout_passes=False),
    )(hidden_preactivation_BH)


# speedup vs baseline: 1.1478x; 1.1313x over previous
import jax
import jax.numpy as jnp
import numpy as np
from jax import lax
from jax.experimental import pallas as pl
from jax.experimental.pallas import tpu as pltpu
from jax.experimental.pallas import tpu_sc as plsc

K = 64
B = 128
H = 32768
NC, NS, L = 2, 16, 16
NW = NC * NS
ROWS_PER_W = B // NW
NVREG = H // L
SEED_VREGS = 64
CHUNK = 124
NCHUNK = (NVREG - SEED_VREGS) // CHUNK
PL_REBUILD = 64
PLCAP = 192
WCAP = 64

INT_MIN = np.int32(-2147483648)
INT_MAX = np.int32(2147483647)
MANT = np.int32(0x7FFFFFFF)

parallel_loop = plsc.parallel_loop


def _iota():
    return lax.iota(jnp.int32, L)


def _keys_of(v):
    u = plsc.bitcast(v, jnp.int32)
    return u ^ ((u >> 31) & MANT)


def _axidx(name):
    return lax.axis_index(name)


def _body(x_hbm, o_hbm, buf0, buf1, zbuf, winbuf, cand_i, cand_k, cand_j,
          sem_in, sem_out):
    wid = _axidx("c") * NS + _axidx("s")
    zeros = jnp.zeros((L,), jnp.float32)
    bases = _iota() * PLCAP
    wbases = _iota() * WCAP
    bufs = [buf0, buf1]

    @parallel_loop(0, H, L, unroll=8, carry=jnp.int32(0))
    def _z(off, c):
        zbuf[pl.ds(off, L)] = zeros
        return c

    def rebuild(buf, cv):
        mvp = jnp.max(cv)

        @parallel_loop(0, mvp, 1, unroll=2, carry=jnp.int32(0))
        def _keys(p, c):
            valid = cv > p
            il = plsc.load_gather(cand_i, [bases + p])
            idx_safe = jnp.where(valid, il, 0)
            v = plsc.load_gather(buf, [idx_safe])
            k = jnp.where(valid, _keys_of(v), INT_MIN)
            cand_k[pl.ds(p * L, L)] = k
            cand_j[pl.ds(p * L, L)] = idx_safe
            return c

        def search_it(_, lohi):
            lo, hi = lohi
            mid = (lo >> 1) + (hi >> 1) + ((lo & 1) | (hi & 1))

            @parallel_loop(0, mvp * L, L, unroll=4,
                           carry=jnp.zeros((L,), jnp.int32))
            def acc(off, a):
                k = cand_k[pl.ds(off, L)]
                return a + (k >= mid).astype(jnp.int32)
            pred = jnp.sum(acc) >= K
            lo = jnp.where(pred, mid, lo)
            hi = jnp.where(pred, hi, mid - jnp.int32(1))
            return lo, hi
        v_key, _ = lax.fori_loop(
            0, 32, search_it, (INT_MIN, jnp.int32(INT_MAX)))

        @parallel_loop(0, mvp * L, L, unroll=4,
                       carry=(jnp.zeros((L,), jnp.int32),
                              jnp.zeros((L,), jnp.int32)))
        def accs(off, a):
            ag, at = a
            k = cand_k[pl.ds(off, L)]
            return ag + (k > v_key).astype(jnp.int32), \
                at + (k == v_key).astype(jnp.int32)
        cnt_gt = jnp.sum(accs[0])
        cnt_tie = jnp.sum(accs[1])
        allow = jnp.int32(K) - cnt_gt

        def tie_search(_):
            def ts_it(_, lohi):
                lo, hi = lohi
                mid = (lo + hi) >> 1

                @parallel_loop(0, mvp * L, L, unroll=4,
                               carry=jnp.zeros((L,), jnp.int32))
                def ta(off, a):
                    k = cand_k[pl.ds(off, L)]
                    ix = cand_j[pl.ds(off, L)]
                    return a + ((k == v_key) & (ix <= mid)).astype(jnp.int32)
                pred = jnp.sum(ta) >= allow
                lo = jnp.where(pred, lo, mid + jnp.int32(1))
                hi = jnp.where(pred, mid, hi)
                return lo, hi
            lo, _ = lax.fori_loop(
                0, 15, ts_it, (jnp.int32(0), jnp.int32(H - 1)))
            return lo
        idx_cut = lax.cond(
            cnt_tie <= allow, lambda _: jnp.int32(INT_MAX), tie_search, 0)

        @parallel_loop(0, mvp, 1, unroll=2,
                       carry=jnp.zeros((L,), jnp.int32))
        def ncv(p, nc):
            k = cand_k[pl.ds(p * L, L)]
            ix = cand_j[pl.ds(p * L, L)]
            keep = (k > v_key) | ((k == v_key) & (ix <= idx_cut))
            nc = nc + keep.astype(jnp.int32)
            plsc.store_scatter(cand_i, [bases + nc - 1], ix, mask=keep)
            return nc

        thr_bits = jnp.full((L,), v_key ^ ((v_key >> 31) & MANT), jnp.int32)
        return ncv, plsc.bitcast(thr_bits, jnp.float32)

    def select_row(buf):
        ninf = jnp.full((L,), -jnp.inf, jnp.float32)

        @parallel_loop(0, SEED_VREGS * L, L, unroll=4,
                       carry=(ninf, ninf, ninf, ninf))
        def tops(off, rs):
            r0, r1, r2, r3 = rs
            v = buf[pl.ds(off, L)]
            h = jnp.maximum(r0, v); v = jnp.minimum(r0, v); r0 = h
            h = jnp.maximum(r1, v); v = jnp.minimum(r1, v); r1 = h
            h = jnp.maximum(r2, v); v = jnp.minimum(r2, v); r2 = h
            r3 = jnp.maximum(r3, v)
            return r0, r1, r2, r3
        thr = jnp.full((L,), jnp.min(tops[3]), jnp.float32)

        @parallel_loop(0, SEED_VREGS * L, L, unroll=8,
                       carry=jnp.zeros((L,), jnp.int32))
        def cv(off, cv):
            v = buf[pl.ds(off, L)]
            m = v >= thr
            cv = cv + m.astype(jnp.int32)
            plsc.store_scatter(
                cand_i, [bases + cv - 1], _iota() + off, mask=m)
            return cv

        def chunk_c(c, carry):
            cv, thr = carry
            base = SEED_VREGS + c * CHUNK

            @parallel_loop(base * L, (base + CHUNK) * L, L, unroll=8,
                           carry=cv)
            def cv(off, cv):
                v = buf[pl.ds(off, L)]
                m = v > thr
                cv = cv + m.astype(jnp.int32)
                plsc.store_scatter(
                    cand_i, [bases + cv - 1], _iota() + off, mask=m)
                return cv

            cv, thr = lax.cond(
                jnp.max(cv) > PL_REBUILD,
                lambda a: rebuild(buf, a[0]), lambda a: a, (cv, thr))
            return cv, thr
        cv, _ = lax.fori_loop(0, NCHUNK, chunk_c, (cv, thr))

        wcv, _ = rebuild(buf, cv)
        return wcv

    base_row = wid * ROWS_PER_W
    pltpu.async_copy(x_hbm.at[base_row], bufs[0], sem_in)
    pwcv = jnp.zeros((L,), jnp.int32)
    for t in range(ROWS_PER_W):
        buf = bufs[t % 2]
        row = base_row + t
        pltpu.make_async_copy(x_hbm.at[row], buf, sem_in).wait()
        if t + 1 < ROWS_PER_W:
            pltpu.async_copy(x_hbm.at[row + 1], bufs[(t + 1) % 2], sem_in)

        wcv = select_row(buf)

        if t >= 1:
            pltpu.make_async_copy(zbuf, o_hbm.at[row - 1], sem_out).wait()

            @parallel_loop(0, jnp.max(pwcv), 1, carry=jnp.int32(0))
            def _rz(p, c):
                valid = pwcv > p
                il = plsc.load_gather(winbuf, [wbases + p])
                plsc.store_scatter(zbuf, [il], zeros, mask=valid)
                return c

        @parallel_loop(0, jnp.max(wcv), 1, carry=jnp.int32(0))
        def _win(p, c):
            valid = wcv > p
            il = plsc.load_gather(cand_i, [bases + p])
            ils = jnp.where(valid, il, 0)
            vals = plsc.load_gather(buf, [ils])
            plsc.store_scatter(zbuf, [ils], vals, mask=valid)
            plsc.store_scatter(winbuf, [wbases + p], ils, mask=valid)
            return c
        pwcv = wcv
        pltpu.async_copy(zbuf, o_hbm.at[row], sem_out)
    pltpu.make_async_copy(
        zbuf, o_hbm.at[base_row + ROWS_PER_W - 1], sem_out).wait()


@jax.jit
def kernel(hidden_preactivation_BH):
    mesh = plsc.VectorSubcoreMesh(
        core_axis_name="c", subcore_axis_name="s",
        num_cores=NC, num_subcores=NS)
    return pl.kernel(
        _body,
        out_type=jax.ShapeDtypeStruct((B, H), jnp.float32),
        mesh=mesh,
        scratch_types=[
            pltpu.VMEM((H,), jnp.float32),
            pltpu.VMEM((H,), jnp.float32),
            pltpu.VMEM((H,), jnp.float32),
            pltpu.VMEM((L * WCAP,), jnp.int32),
            pltpu.VMEM((L * PLCAP,), jnp.int32),
            pltpu.VMEM((L * PLCAP,), jnp.int32),
            pltpu.VMEM((L * PLCAP,), jnp.int32),
            pltpu.SemaphoreType.DMA,
            pltpu.SemaphoreType.DMA,
        ],
        compiler_params=pltpu.CompilerParams(needs_layout_passes=False),
    )(hidden_preactivation_BH)


# speedup vs baseline: 1.1740x; 1.0229x over previous
import jax
import jax.numpy as jnp
import numpy as np
from jax import lax
from jax.experimental import pallas as pl
from jax.experimental.pallas import tpu as pltpu
from jax.experimental.pallas import tpu_sc as plsc

K = 64
B = 128
H = 32768
NC, NS, L = 2, 16, 16
NW = NC * NS
ROWS_PER_W = B // NW
NVREG = H // L
SEED_VREGS = 64
CHUNK = 124
NCHUNK = (NVREG - SEED_VREGS) // CHUNK
PL_REBUILD = 64
PLCAP = 192
WCAP = 64

INT_MIN = np.int32(-2147483648)
INT_MAX = np.int32(2147483647)
MANT = np.int32(0x7FFFFFFF)

parallel_loop = plsc.parallel_loop


def _iota():
    return lax.iota(jnp.int32, L)


def _keys_of(v):
    u = plsc.bitcast(v, jnp.int32)
    return u ^ ((u >> 31) & MANT)


def _axidx(name):
    return lax.axis_index(name)


def _body(x_hbm, o_hbm, buf0, buf1, zbuf, winbuf, cand_i, cand_k, cand_j,
          sem_in, sem_out):
    wid = _axidx("c") * NS + _axidx("s")
    zeros = jnp.zeros((L,), jnp.float32)
    bases = _iota() * PLCAP
    wbases = _iota() * WCAP
    bufs = [buf0, buf1]

    @parallel_loop(0, H, L, unroll=8, carry=jnp.int32(0))
    def _z(off, c):
        zbuf[pl.ds(off, L)] = zeros
        return c

    def rebuild(buf, cv):
        mvp = jnp.max(cv)

        @parallel_loop(0, mvp, 1, unroll=2, carry=jnp.int32(0))
        def _keys(p, c):
            valid = cv > p
            il = plsc.load_gather(cand_i, [bases + p])
            idx_safe = jnp.where(valid, il, 0)
            v = plsc.load_gather(buf, [idx_safe])
            k = jnp.where(valid, _keys_of(v), INT_MIN)
            cand_k[pl.ds(p * L, L)] = k
            cand_j[pl.ds(p * L, L)] = idx_safe
            return c

        def search_it(_, lohi):
            lo, hi = lohi
            mid = (lo >> 1) + (hi >> 1) + ((lo & 1) | (hi & 1))

            @parallel_loop(0, mvp * L, L, unroll=4,
                           carry=jnp.zeros((L,), jnp.int32))
            def acc(off, a):
                k = cand_k[pl.ds(off, L)]
                return a + (k >= mid).astype(jnp.int32)
            pred = jnp.sum(acc) >= K
            lo = jnp.where(pred, mid, lo)
            hi = jnp.where(pred, hi, mid - jnp.int32(1))
            return lo, hi
        v_key, _ = lax.fori_loop(
            0, 32, search_it, (INT_MIN, jnp.int32(INT_MAX)))

        @parallel_loop(0, mvp * L, L, unroll=4,
                       carry=(jnp.zeros((L,), jnp.int32),
                              jnp.zeros((L,), jnp.int32)))
        def accs(off, a):
            ag, at = a
            k = cand_k[pl.ds(off, L)]
            return ag + (k > v_key).astype(jnp.int32), \
                at + (k == v_key).astype(jnp.int32)
        cnt_gt = jnp.sum(accs[0])
        cnt_tie = jnp.sum(accs[1])
        allow = jnp.int32(K) - cnt_gt

        def tie_search(_):
            def ts_it(_, lohi):
                lo, hi = lohi
                mid = (lo + hi) >> 1

                @parallel_loop(0, mvp * L, L, unroll=4,
                               carry=jnp.zeros((L,), jnp.int32))
                def ta(off, a):
                    k = cand_k[pl.ds(off, L)]
                    ix = cand_j[pl.ds(off, L)]
                    return a + ((k == v_key) & (ix <= mid)).astype(jnp.int32)
                pred = jnp.sum(ta) >= allow
                lo = jnp.where(pred, lo, mid + jnp.int32(1))
                hi = jnp.where(pred, mid, hi)
                return lo, hi
            lo, _ = lax.fori_loop(
                0, 15, ts_it, (jnp.int32(0), jnp.int32(H - 1)))
            return lo
        idx_cut = lax.cond(
            cnt_tie <= allow, lambda _: jnp.int32(INT_MAX), tie_search, 0)

        @parallel_loop(0, mvp, 1, unroll=2,
                       carry=jnp.zeros((L,), jnp.int32))
        def ncv(p, nc):
            k = cand_k[pl.ds(p * L, L)]
            ix = cand_j[pl.ds(p * L, L)]
            keep = (k > v_key) | ((k == v_key) & (ix <= idx_cut))
            nc = nc + keep.astype(jnp.int32)
            plsc.store_scatter(cand_i, [bases + nc - 1], ix, mask=keep)
            return nc

        thr_bits = jnp.full((L,), v_key ^ ((v_key >> 31) & MANT), jnp.int32)
        return ncv, plsc.bitcast(thr_bits, jnp.float32)

    def select_row(buf):
        ninf = jnp.full((L,), -jnp.inf, jnp.float32)

        @parallel_loop(0, SEED_VREGS * L, L, unroll=4,
                       carry=(ninf, ninf, ninf, ninf))
        def tops(off, rs):
            r0, r1, r2, r3 = rs
            v = buf[pl.ds(off, L)]
            h = jnp.maximum(r0, v); v = jnp.minimum(r0, v); r0 = h
            h = jnp.maximum(r1, v); v = jnp.minimum(r1, v); r1 = h
            h = jnp.maximum(r2, v); v = jnp.minimum(r2, v); r2 = h
            r3 = jnp.maximum(r3, v)
            return r0, r1, r2, r3
        thr = jnp.full((L,), jnp.min(tops[3]), jnp.float32)

        @parallel_loop(0, SEED_VREGS * L, L, unroll=8, carry=bases - 1)
        def dm(off, dm):
            v = buf[pl.ds(off, L)]
            m = v >= thr
            dm = dm + m.astype(jnp.int32)
            plsc.store_scatter(cand_i, [dm], _iota() + off, mask=m)
            return dm

        def chunk_c(c, carry):
            dm, thr = carry
            base = SEED_VREGS + c * CHUNK

            @parallel_loop(base * L, (base + CHUNK) * L, L, unroll=8,
                           carry=dm)
            def dm(off, dm):
                v = buf[pl.ds(off, L)]
                m = v > thr
                dm = dm + m.astype(jnp.int32)
                plsc.store_scatter(cand_i, [dm], _iota() + off, mask=m)
                return dm

            def do_rb(a):
                ncv, thr = rebuild(buf, a[0] - bases + 1)
                return bases + ncv - 1, thr

            dm, thr = lax.cond(
                jnp.max(dm - bases) + 1 > PL_REBUILD,
                do_rb, lambda a: a, (dm, thr))
            return dm, thr
        dm, _ = lax.fori_loop(0, NCHUNK, chunk_c, (dm, thr))
        cv = dm - bases + 1

        wcv, _ = rebuild(buf, cv)
        return wcv

    base_row = wid * ROWS_PER_W
    pltpu.async_copy(x_hbm.at[base_row], bufs[0], sem_in)
    pwcv = jnp.zeros((L,), jnp.int32)
    for t in range(ROWS_PER_W):
        buf = bufs[t % 2]
        row = base_row + t
        pltpu.make_async_copy(x_hbm.at[row], buf, sem_in).wait()
        if t + 1 < ROWS_PER_W:
            pltpu.async_copy(x_hbm.at[row + 1], bufs[(t + 1) % 2], sem_in)

        wcv = select_row(buf)

        if t >= 1:
            pltpu.make_async_copy(zbuf, o_hbm.at[row - 1], sem_out).wait()

            @parallel_loop(0, jnp.max(pwcv), 1, carry=jnp.int32(0))
            def _rz(p, c):
                valid = pwcv > p
                il = plsc.load_gather(winbuf, [wbases + p])
                plsc.store_scatter(zbuf, [il], zeros, mask=valid)
                return c

        @parallel_loop(0, jnp.max(wcv), 1, carry=jnp.int32(0))
        def _win(p, c):
            valid = wcv > p
            il = plsc.load_gather(cand_i, [bases + p])
            ils = jnp.where(valid, il, 0)
            vals = plsc.load_gather(buf, [ils])
            plsc.store_scatter(zbuf, [ils], vals, mask=valid)
            plsc.store_scatter(winbuf, [wbases + p], ils, mask=valid)
            return c
        pwcv = wcv
        pltpu.async_copy(zbuf, o_hbm.at[row], sem_out)
    pltpu.make_async_copy(
        zbuf, o_hbm.at[base_row + ROWS_PER_W - 1], sem_out).wait()


@jax.jit
def kernel(hidden_preactivation_BH):
    mesh = plsc.VectorSubcoreMesh(
        core_axis_name="c", subcore_axis_name="s",
        num_cores=NC, num_subcores=NS)
    return pl.kernel(
        _body,
        out_type=jax.ShapeDtypeStruct((B, H), jnp.float32),
        mesh=mesh,
        scratch_types=[
            pltpu.VMEM((H,), jnp.float32),
            pltpu.VMEM((H,), jnp.float32),
            pltpu.VMEM((H,), jnp.float32),
            pltpu.VMEM((L * WCAP,), jnp.int32),
            pltpu.VMEM((L * PLCAP,), jnp.int32),
            pltpu.VMEM((L * PLCAP,), jnp.int32),
            pltpu.VMEM((L * PLCAP,), jnp.int32),
            pltpu.SemaphoreType.DMA,
            pltpu.SemaphoreType.DMA,
        ],
        compiler_params=pltpu.CompilerParams(needs_layout_passes=False),
    )(hidden_preactivation_BH)
